# baseline (speedup 1.0000x reference)
import functools

import jax
import jax.numpy as jnp
from jax import lax
from jax.experimental import pallas as pl
from jax.experimental.pallas import tpu as pltpu
from jax.experimental.pallas import tpu_sc as plsc

N_DR_ = 25000
N_DI_ = 25000
E_ = 400000
D_ = 128
H_ = 8
B_ = 16384

_EB = 512
_NBLK = 896
_BPW = _NBLK // 32
_ZCH = 112
_NBUF = 4


def _segsum_call(n_pad, w, n_chunks, with_deg, src3, dst3, tables):
    C = n_chunks + (1 if with_deg else 0)
    rows_per = n_pad // 16
    assert rows_per % _ZCH == 0
    mesh = plsc.VectorSubcoreMesh(core_axis_name="c", subcore_axis_name="s")

    @functools.partial(
        pl.kernel,
        mesh=mesh,
        compiler_params=pltpu.CompilerParams(use_tc_tiling_on_sc=False),
        out_type=jax.ShapeDtypeStruct((2, C, n_pad, w), jnp.float32),
        scratch_types=[
            pltpu.VMEM((_BPW, _EB), jnp.int32),
            pltpu.VMEM((_BPW, _EB), jnp.int32),
            pltpu.VMEM((_NBUF, _EB, w), jnp.float32),
            pltpu.VMEM((_ZCH, w), jnp.float32),
            pltpu.VMEM((_EB, w), jnp.float32),
            pltpu.VMEM_SHARED((n_pad, w), jnp.float32),
            pltpu.SemaphoreType.DMA((_NBUF,)),
            pltpu.SemaphoreType.DMA((_NBUF,)),
        ],
    )
    def k(src_h, dst_h, *rest):
        tabs = rest[:n_chunks]
        zrow_h = rest[n_chunks]
        ones_h = rest[n_chunks + 1]
        out_h = rest[n_chunks + 2]
        src_v, dst_v, rows_v, zbuf, obuf, acc, gsem, ssem = rest[n_chunks + 3:]
        cid = lax.axis_index("c")
        sid = lax.axis_index("s")
        wid = cid * 16 + sid
        pltpu.sync_copy(src_h.at[wid], src_v)
        pltpu.sync_copy(dst_h.at[wid], dst_v)
        pltpu.sync_copy(zrow_h, zbuf)
        pltpu.sync_copy(ones_h, obuf)
        r0 = sid * rows_per
        for c in range(C):
            @pl.loop(0, rows_per, step=_ZCH)
            def _(rz):
                pltpu.sync_copy(zbuf, acc.at[pl.ds(r0 + rz, _ZCH)])
            plsc.subcore_barrier()
            if c < n_chunks:
                def _g_start(b, i):
                    pltpu.async_copy(tabs[c].at[src_v.at[b]],
                                     rows_v.at[i], gsem.at[i])

                def _g_wait(b, i):
                    pltpu.make_async_copy(tabs[c].at[src_v.at[b]],
                                          rows_v.at[i], gsem.at[i]).wait()

                def _s_start(b, i):
                    pltpu.async_copy(rows_v.at[i], acc.at[dst_v.at[b]],
                                     ssem.at[i], add=True)

                def _s_wait(b, i):
                    pltpu.make_async_copy(rows_v.at[i], acc.at[dst_v.at[b]],
                                          ssem.at[i]).wait()

                for i in range(_NBUF):
                    _g_start(i, i)

                @pl.loop(0, _BPW, step=_NBUF)
                def _(g):
                    for i in range(_NBUF):
                        _g_wait(g + i, i)
                        _s_start(g + i, i)
                    for i in range(_NBUF):
                        _s_wait(g + i, i)
                        nb = jnp.minimum(g + _NBUF + i, _BPW - 1)
                        _g_start(nb, i)
                for i in range(_NBUF):
                    _g_wait(_BPW - 1, i)
            else:
                def _d_start(b, i):
                    pltpu.async_copy(obuf, acc.at[dst_v.at[b]],
                                     ssem.at[i], add=True)

                def _d_wait(b, i):
                    pltpu.make_async_copy(obuf, acc.at[dst_v.at[b]],
                                          ssem.at[i]).wait()

                for i in range(_NBUF):
                    _d_start(i, i)

                @pl.loop(_NBUF, _BPW, step=_NBUF)
                def _(g):
                    for i in range(_NBUF):
                        _d_wait(g - _NBUF + i, i)
                        _d_start(g + i, i)
                for i in range(_NBUF):
                    _d_wait(_BPW - _NBUF + i, i)
            plsc.subcore_barrier()
            pltpu.sync_copy(acc.at[pl.ds(r0, rows_per)],
                            out_h.at[cid, c, pl.ds(r0, rows_per)])
            plsc.subcore_barrier()

    zrow = jnp.zeros((_ZCH, w), jnp.float32)
    ones = jnp.ones((_EB, w), jnp.float32)
    return k(src3, dst3, *tables, zrow, ones)


def _pad_edges(e, n):
    pad = jnp.full((2, _NBLK * _EB - E_), n, jnp.int32)
    return jnp.concatenate([e, pad], axis=1).reshape(2, 32, _BPW, _EB)


def _chunk_table(hw, n_pad, w):
    n = hw.shape[0]
    hwp = jnp.pad(hw, ((0, n_pad - n), (0, 0)))
    return [hwp[:, i * w:(i + 1) * w] for i in range(D_ // w)]


def _graph_pass(edges3, hw, n, n_pad, w, deg=None):
    tables = _chunk_table(hw, n_pad, w)
    with_deg = deg is None
    parts = _segsum_call(n_pad, w, len(tables), with_deg,
                         edges3[0], edges3[1], tables)
    sums = parts[0] + parts[1]
    agg = jnp.concatenate([sums[c] for c in range(len(tables))], axis=1)
    if with_deg:
        deg = sums[len(tables), :, 0]
    res = jax.nn.relu(agg / jnp.maximum(deg, 1.0)[:, None])
    return res, deg


def _self_att(x, Wq, bq, Wk, bk):
    Bn, M, Cc = x.shape
    Dh = Cc // H_
    q = (jnp.mean(x, axis=1) @ Wq + bq).reshape(Bn, 1, H_, Dh).transpose(0, 2, 1, 3)
    k = (x @ Wk + bk).reshape(Bn, M, H_, Dh).transpose(0, 2, 3, 1)
    v = x.reshape(Bn, M, H_, Dh).transpose(0, 2, 1, 3)
    alpha = jax.nn.softmax((q @ k) / (float(Dh) ** 0.5), axis=-1)
    o = alpha @ v
    return o.transpose(0, 2, 1, 3).reshape(Bn, H_ * Dh)


def _rotate(a, b):
    a_re, a_im = jnp.split(a, 2, axis=-1)
    b_re, b_im = jnp.split(b, 2, axis=-1)
    return jnp.concatenate([a_re * b_re - a_im * b_im,
                            a_re * b_im + a_im * b_re], axis=-1)


def kernel(drdr_similarity_graph, didi_similarity_graph, drdr_dissimilarity_graph, didi_dissimilarity_graph, positive_heterograph, negative_heterograph, drug_feature, disease_feature, sample, emb_dr, emb_di, W_gt_dr, b_gt_dr, W_gt_di, b_gt_di, W_drug_lin, b_drug_lin, W_dis_lin, b_dis_lin, W_hgt, b_hgt, Wq_dr, bq_dr, Wk_dr, bk_dr, Wq_di, bq_di, Wk_di, bk_di, W1, b1, W2, b2, W3, b3, W4, b4):
    n1, n1p, w1 = N_DR_, 25088, 16
    n2, n2p, w2 = N_DR_ + N_DI_, 50176, 16

    hw_dr = emb_dr @ W_gt_dr + b_gt_dr
    hw_di = emb_di @ W_gt_di + b_gt_di

    e_drdr_s = _pad_edges(drdr_similarity_graph, n1)
    e_drdr_d = _pad_edges(drdr_dissimilarity_graph, n1)
    e_didi_s = _pad_edges(didi_similarity_graph, n1)
    e_didi_d = _pad_edges(didi_dissimilarity_graph, n1)
    e_pos = _pad_edges(positive_heterograph, n2)
    e_neg = _pad_edges(negative_heterograph, n2)

    dr_sim_p, _ = _graph_pass(e_drdr_s, hw_dr, n1, n1p, w1)
    dr_sim_n, _ = _graph_pass(e_drdr_d, hw_dr, n1, n1p, w1)
    di_sim_p, _ = _graph_pass(e_didi_s, hw_di, n1, n1p, w1)
    di_sim_n, _ = _graph_pass(e_didi_d, hw_di, n1, n1p, w1)

    drug_h = drug_feature @ W_drug_lin + b_drug_lin
    dis_h = disease_feature @ W_dis_lin + b_dis_lin
    feat0 = jnp.concatenate([drug_h, dis_h], axis=0)

    fw0 = feat0 @ W_hgt + b_hgt
    f1p, deg_p = _graph_pass(e_pos, fw0, n2, n2p, w2)
    f1n, deg_n = _graph_pass(e_neg, fw0, n2, n2p, w2)
    fw1p = f1p[:n2] @ W_hgt + b_hgt
    fw1n = f1n[:n2] @ W_hgt + b_hgt
    f2p, _ = _graph_pass(e_pos, fw1p, n2, n2p, w2, deg=deg_p)
    f2n, _ = _graph_pass(e_neg, fw1n, n2, n2p, w2, deg=deg_n)

    dr = jnp.stack([dr_sim_p[:n1], dr_sim_n[:n1],
                    f2p[:N_DR_], f2n[:N_DR_]], axis=1)
    di = jnp.stack([di_sim_p[:n1], di_sim_n[:n1],
                    f2p[N_DR_:n2], f2n[N_DR_:n2]], axis=1)
    dr_final = _self_att(dr, Wq_dr, bq_dr, Wk_dr, bk_dr)
    di_final = _self_att(di, Wq_di, bq_di, Wk_di, bk_di)

    dr_s = jnp.take(dr_final, sample[:, 0], axis=0)
    di_s = jnp.take(di_final, sample[:, 1], axis=0)
    m_result = dr_s * di_s
    r_result = _rotate(dr_s, di_s)
    drdi = jnp.concatenate([dr_s, di_s, m_result, r_result], axis=1)
    h = jax.nn.relu(drdi @ W1 + b1)
    h = jax.nn.relu(h @ W2 + b2)
    h = jax.nn.relu(h @ W3 + b3)
    return h @ W4 + b4


# speedup vs baseline: 2.5662x; 2.5662x over previous
import functools

import jax
import jax.numpy as jnp
from jax import lax
from jax.experimental import pallas as pl
from jax.experimental.pallas import tpu as pltpu
from jax.experimental.pallas import tpu_sc as plsc

N_DR_ = 25000
N_DI_ = 25000
E_ = 400000
D_ = 128
H_ = 8
B_ = 16384

_EB = 128
_NBLK = 3200
_BPW = _NBLK // 32
_ZCH = 112
_NBUF = 4


def _segsum_call(n_pad, w, n_chunks, with_deg, src3, dst3, tables):
    C = n_chunks + (1 if with_deg else 0)
    rows_per = n_pad // 16
    assert rows_per % _ZCH == 0
    mesh = plsc.VectorSubcoreMesh(core_axis_name="c", subcore_axis_name="s")

    @functools.partial(
        pl.kernel,
        mesh=mesh,
        compiler_params=pltpu.CompilerParams(use_tc_tiling_on_sc=False),
        out_type=jax.ShapeDtypeStruct((2, C, n_pad, w), jnp.float32),
        scratch_types=[
            pltpu.VMEM((_BPW, _EB), jnp.int32),
            pltpu.VMEM((_BPW, _EB), jnp.int32),
            pltpu.VMEM((_NBUF, _EB, w), jnp.float32),
            pltpu.VMEM((_ZCH, w), jnp.float32),
            pltpu.VMEM((_EB, w), jnp.float32),
            pltpu.VMEM_SHARED((n_pad, w), jnp.float32),
            pltpu.SemaphoreType.DMA((_NBUF,)),
            pltpu.SemaphoreType.DMA((_NBUF,)),
        ],
    )
    def k(src_h, dst_h, *rest):
        tabs = rest[:n_chunks]
        zrow_h = rest[n_chunks]
        ones_h = rest[n_chunks + 1]
        out_h = rest[n_chunks + 2]
        src_v, dst_v, rows_v, zbuf, obuf, acc, gsem, ssem = rest[n_chunks + 3:]
        cid = lax.axis_index("c")
        sid = lax.axis_index("s")
        wid = cid * 16 + sid
        pltpu.sync_copy(src_h.at[wid], src_v)
        pltpu.sync_copy(dst_h.at[wid], dst_v)
        pltpu.sync_copy(zrow_h, zbuf)
        pltpu.sync_copy(ones_h, obuf)
        r0 = sid * rows_per
        for c in range(C):
            @pl.loop(0, rows_per, step=_ZCH)
            def _(rz):
                pltpu.sync_copy(zbuf, acc.at[pl.ds(r0 + rz, _ZCH)])
            plsc.subcore_barrier()
            if c < n_chunks:
                def _g_start(b, i):
                    pltpu.async_copy(tabs[c].at[src_v.at[b]],
                                     rows_v.at[i], gsem.at[i])

                def _g_wait(b, i):
                    pltpu.make_async_copy(tabs[c].at[src_v.at[b]],
                                          rows_v.at[i], gsem.at[i]).wait()

                def _s_start(b, i):
                    pltpu.async_copy(rows_v.at[i], acc.at[dst_v.at[b]],
                                     ssem.at[i], add=True)

                def _s_wait(b, i):
                    pltpu.make_async_copy(rows_v.at[i], acc.at[dst_v.at[b]],
                                          ssem.at[i]).wait()

                for i in range(_NBUF):
                    _g_start(i, i)

                @pl.loop(0, _BPW, step=_NBUF)
                def _(g):
                    for i in range(_NBUF):
                        _g_wait(g + i, i)
                        _s_start(g + i, i)
                    for i in range(_NBUF):
                        _s_wait(g + i, i)
                        nb = jnp.minimum(g + _NBUF + i, _BPW - 1)
                        _g_start(nb, i)
                for i in range(_NBUF):
                    _g_wait(_BPW - 1, i)
            else:
                def _d_start(b, i):
                    pltpu.async_copy(obuf, acc.at[dst_v.at[b]],
                                     ssem.at[i], add=True)

                def _d_wait(b, i):
                    pltpu.make_async_copy(obuf, acc.at[dst_v.at[b]],
                                          ssem.at[i]).wait()

                for i in range(_NBUF):
                    _d_start(i, i)

                @pl.loop(_NBUF, _BPW, step=_NBUF)
                def _(g):
                    for i in range(_NBUF):
                        _d_wait(g - _NBUF + i, i)
                        _d_start(g + i, i)
                for i in range(_NBUF):
                    _d_wait(_BPW - _NBUF + i, i)
            plsc.subcore_barrier()
            pltpu.sync_copy(acc.at[pl.ds(r0, rows_per)],
                            out_h.at[cid, c, pl.ds(r0, rows_per)])
            plsc.subcore_barrier()

    zrow = jnp.zeros((_ZCH, w), jnp.float32)
    ones = jnp.ones((_EB, w), jnp.float32)
    return k(src3, dst3, *tables, zrow, ones)


def _pad_edges(e, n):
    pad = jnp.full((2, _NBLK * _EB - E_), n, jnp.int32)
    return jnp.concatenate([e, pad], axis=1).reshape(2, 32, _BPW, _EB)


def _chunk_table(hw, n_pad, w):
    n = hw.shape[0]
    hwp = jnp.pad(hw, ((0, n_pad - n), (0, 0)))
    return [hwp[:, i * w:(i + 1) * w] for i in range(D_ // w)]


def _graph_pass(edges3, hw, n, n_pad, w, deg=None):
    tables = _chunk_table(hw, n_pad, w)
    with_deg = deg is None
    parts = _segsum_call(n_pad, w, len(tables), with_deg,
                         edges3[0], edges3[1], tables)
    sums = parts[0] + parts[1]
    agg = jnp.concatenate([sums[c] for c in range(len(tables))], axis=1)
    if with_deg:
        deg = sums[len(tables), :, 0]
    res = jax.nn.relu(agg / jnp.maximum(deg, 1.0)[:, None])
    return res, deg


def _self_att(x, Wq, bq, Wk, bk):
    Bn, M, Cc = x.shape
    Dh = Cc // H_
    q = (jnp.mean(x, axis=1) @ Wq + bq).reshape(Bn, 1, H_, Dh).transpose(0, 2, 1, 3)
    k = (x @ Wk + bk).reshape(Bn, M, H_, Dh).transpose(0, 2, 3, 1)
    v = x.reshape(Bn, M, H_, Dh).transpose(0, 2, 1, 3)
    alpha = jax.nn.softmax((q @ k) / (float(Dh) ** 0.5), axis=-1)
    o = alpha @ v
    return o.transpose(0, 2, 1, 3).reshape(Bn, H_ * Dh)


def _rotate(a, b):
    a_re, a_im = jnp.split(a, 2, axis=-1)
    b_re, b_im = jnp.split(b, 2, axis=-1)
    return jnp.concatenate([a_re * b_re - a_im * b_im,
                            a_re * b_im + a_im * b_re], axis=-1)


def kernel(drdr_similarity_graph, didi_similarity_graph, drdr_dissimilarity_graph, didi_dissimilarity_graph, positive_heterograph, negative_heterograph, drug_feature, disease_feature, sample, emb_dr, emb_di, W_gt_dr, b_gt_dr, W_gt_di, b_gt_di, W_drug_lin, b_drug_lin, W_dis_lin, b_dis_lin, W_hgt, b_hgt, Wq_dr, bq_dr, Wk_dr, bk_dr, Wq_di, bq_di, Wk_di, bk_di, W1, b1, W2, b2, W3, b3, W4, b4):
    n1, n1p, w1 = N_DR_, 25088, 16
    n2, n2p, w2 = N_DR_ + N_DI_, 50176, 16

    hw_dr = emb_dr @ W_gt_dr + b_gt_dr
    hw_di = emb_di @ W_gt_di + b_gt_di

    e_drdr_s = _pad_edges(drdr_similarity_graph, n1)
    e_drdr_d = _pad_edges(drdr_dissimilarity_graph, n1)
    e_didi_s = _pad_edges(didi_similarity_graph, n1)
    e_didi_d = _pad_edges(didi_dissimilarity_graph, n1)
    e_pos = _pad_edges(positive_heterograph, n2)
    e_neg = _pad_edges(negative_heterograph, n2)

    dr_sim_p, _ = _graph_pass(e_drdr_s, hw_dr, n1, n1p, w1)
    dr_sim_n, _ = _graph_pass(e_drdr_d, hw_dr, n1, n1p, w1)
    di_sim_p, _ = _graph_pass(e_didi_s, hw_di, n1, n1p, w1)
    di_sim_n, _ = _graph_pass(e_didi_d, hw_di, n1, n1p, w1)

    drug_h = drug_feature @ W_drug_lin + b_drug_lin
    dis_h = disease_feature @ W_dis_lin + b_dis_lin
    feat0 = jnp.concatenate([drug_h, dis_h], axis=0)

    fw0 = feat0 @ W_hgt + b_hgt
    f1p, deg_p = _graph_pass(e_pos, fw0, n2, n2p, w2)
    f1n, deg_n = _graph_pass(e_neg, fw0, n2, n2p, w2)
    fw1p = f1p[:n2] @ W_hgt + b_hgt
    fw1n = f1n[:n2] @ W_hgt + b_hgt
    f2p, _ = _graph_pass(e_pos, fw1p, n2, n2p, w2, deg=deg_p)
    f2n, _ = _graph_pass(e_neg, fw1n, n2, n2p, w2, deg=deg_n)

    dr = jnp.stack([dr_sim_p[:n1], dr_sim_n[:n1],
                    f2p[:N_DR_], f2n[:N_DR_]], axis=1)
    di = jnp.stack([di_sim_p[:n1], di_sim_n[:n1],
                    f2p[N_DR_:n2], f2n[N_DR_:n2]], axis=1)
    dr_final = _self_att(dr, Wq_dr, bq_dr, Wk_dr, bk_dr)
    di_final = _self_att(di, Wq_di, bq_di, Wk_di, bk_di)

    dr_s = jnp.take(dr_final, sample[:, 0], axis=0)
    di_s = jnp.take(di_final, sample[:, 1], axis=0)
    m_result = dr_s * di_s
    r_result = _rotate(dr_s, di_s)
    drdi = jnp.concatenate([dr_s, di_s, m_result, r_result], axis=1)
    h = jax.nn.relu(drdi @ W1 + b1)
    h = jax.nn.relu(h @ W2 + b2)
    h = jax.nn.relu(h @ W3 + b3)
    return h @ W4 + b4


# speedup vs baseline: 2.7931x; 1.0884x over previous
import functools

import jax
import jax.numpy as jnp
from jax import lax
from jax.experimental import pallas as pl
from jax.experimental.pallas import tpu as pltpu
from jax.experimental.pallas import tpu_sc as plsc

N_DR_ = 25000
N_DI_ = 25000
E_ = 400000
D_ = 128
H_ = 8
B_ = 16384

_EB = 128
_NBLK = 3200
_BPW = _NBLK // 32
_ZCH = 112
_NBUF = 10


def _segsum_call(n_pad, w, n_chunks, with_deg, src3, dst3, tables):
    C = n_chunks + (1 if with_deg else 0)
    rows_per = n_pad // 16
    assert rows_per % _ZCH == 0
    mesh = plsc.VectorSubcoreMesh(core_axis_name="c", subcore_axis_name="s")

    @functools.partial(
        pl.kernel,
        mesh=mesh,
        compiler_params=pltpu.CompilerParams(use_tc_tiling_on_sc=False),
        out_type=jax.ShapeDtypeStruct((2, C, n_pad, w), jnp.float32),
        scratch_types=[
            pltpu.VMEM((_BPW, _EB), jnp.int32),
            pltpu.VMEM((_BPW, _EB), jnp.int32),
            pltpu.VMEM((_NBUF, _EB, w), jnp.float32),
            pltpu.VMEM((_ZCH, w), jnp.float32),
            pltpu.VMEM((_EB, w), jnp.float32),
            pltpu.VMEM_SHARED((n_pad, w), jnp.float32),
            pltpu.SemaphoreType.DMA((_NBUF,)),
            pltpu.SemaphoreType.DMA((_NBUF,)),
        ],
    )
    def k(src_h, dst_h, *rest):
        tabs = rest[:n_chunks]
        zrow_h = rest[n_chunks]
        ones_h = rest[n_chunks + 1]
        out_h = rest[n_chunks + 2]
        src_v, dst_v, rows_v, zbuf, obuf, acc, gsem, ssem = rest[n_chunks + 3:]
        cid = lax.axis_index("c")
        sid = lax.axis_index("s")
        wid = cid * 16 + sid
        pltpu.sync_copy(src_h.at[wid], src_v)
        pltpu.sync_copy(dst_h.at[wid], dst_v)
        pltpu.sync_copy(zrow_h, zbuf)
        pltpu.sync_copy(ones_h, obuf)
        r0 = sid * rows_per
        for c in range(C):
            @pl.loop(0, rows_per, step=_ZCH)
            def _(rz):
                pltpu.sync_copy(zbuf, acc.at[pl.ds(r0 + rz, _ZCH)])
            plsc.subcore_barrier()
            if c < n_chunks:
                def _g_start(b, i):
                    pltpu.async_copy(tabs[c].at[src_v.at[b]],
                                     rows_v.at[i], gsem.at[i])

                def _g_wait(b, i):
                    pltpu.make_async_copy(tabs[c].at[src_v.at[b]],
                                          rows_v.at[i], gsem.at[i]).wait()

                def _s_start(b, i):
                    pltpu.async_copy(rows_v.at[i], acc.at[dst_v.at[b]],
                                     ssem.at[i], add=True)

                def _s_wait(b, i):
                    pltpu.make_async_copy(rows_v.at[i], acc.at[dst_v.at[b]],
                                          ssem.at[i]).wait()

                for i in range(_NBUF):
                    _g_start(i, i)

                @pl.loop(0, _BPW, step=_NBUF)
                def _(g):
                    for i in range(_NBUF):
                        _g_wait(g + i, i)
                        _s_start(g + i, i)
                    for i in range(_NBUF):
                        _s_wait(g + i, i)
                        nb = jnp.minimum(g + _NBUF + i, _BPW - 1)
                        _g_start(nb, i)
                for i in range(_NBUF):
                    _g_wait(_BPW - 1, i)
            else:
                def _d_start(b, i):
                    pltpu.async_copy(obuf, acc.at[dst_v.at[b]],
                                     ssem.at[i], add=True)

                def _d_wait(b, i):
                    pltpu.make_async_copy(obuf, acc.at[dst_v.at[b]],
                                          ssem.at[i]).wait()

                for i in range(_NBUF):
                    _d_start(i, i)

                @pl.loop(_NBUF, _BPW, step=_NBUF)
                def _(g):
                    for i in range(_NBUF):
                        _d_wait(g - _NBUF + i, i)
                        _d_start(g + i, i)
                for i in range(_NBUF):
                    _d_wait(_BPW - _NBUF + i, i)
            plsc.subcore_barrier()
            pltpu.sync_copy(acc.at[pl.ds(r0, rows_per)],
                            out_h.at[cid, c, pl.ds(r0, rows_per)])
            plsc.subcore_barrier()

    zrow = jnp.zeros((_ZCH, w), jnp.float32)
    ones = jnp.ones((_EB, w), jnp.float32)
    return k(src3, dst3, *tables, zrow, ones)


def _pad_edges(e, n):
    pad = jnp.full((2, _NBLK * _EB - E_), n, jnp.int32)
    return jnp.concatenate([e, pad], axis=1).reshape(2, 32, _BPW, _EB)


def _chunk_table(hw, n_pad, w):
    n = hw.shape[0]
    hwp = jnp.pad(hw, ((0, n_pad - n), (0, 0)))
    return [hwp[:, i * w:(i + 1) * w] for i in range(D_ // w)]


def _graph_pass(edges3, hw, n, n_pad, w, deg=None):
    tables = _chunk_table(hw, n_pad, w)
    with_deg = deg is None
    parts = _segsum_call(n_pad, w, len(tables), with_deg,
                         edges3[0], edges3[1], tables)
    sums = parts[0] + parts[1]
    agg = jnp.concatenate([sums[c] for c in range(len(tables))], axis=1)
    if with_deg:
        deg = sums[len(tables), :, 0]
    res = jax.nn.relu(agg / jnp.maximum(deg, 1.0)[:, None])
    return res, deg


def _self_att(x, Wq, bq, Wk, bk):
    Bn, M, Cc = x.shape
    Dh = Cc // H_
    q = (jnp.mean(x, axis=1) @ Wq + bq).reshape(Bn, 1, H_, Dh).transpose(0, 2, 1, 3)
    k = (x @ Wk + bk).reshape(Bn, M, H_, Dh).transpose(0, 2, 3, 1)
    v = x.reshape(Bn, M, H_, Dh).transpose(0, 2, 1, 3)
    alpha = jax.nn.softmax((q @ k) / (float(Dh) ** 0.5), axis=-1)
    o = alpha @ v
    return o.transpose(0, 2, 1, 3).reshape(Bn, H_ * Dh)


def _rotate(a, b):
    a_re, a_im = jnp.split(a, 2, axis=-1)
    b_re, b_im = jnp.split(b, 2, axis=-1)
    return jnp.concatenate([a_re * b_re - a_im * b_im,
                            a_re * b_im + a_im * b_re], axis=-1)


def kernel(drdr_similarity_graph, didi_similarity_graph, drdr_dissimilarity_graph, didi_dissimilarity_graph, positive_heterograph, negative_heterograph, drug_feature, disease_feature, sample, emb_dr, emb_di, W_gt_dr, b_gt_dr, W_gt_di, b_gt_di, W_drug_lin, b_drug_lin, W_dis_lin, b_dis_lin, W_hgt, b_hgt, Wq_dr, bq_dr, Wk_dr, bk_dr, Wq_di, bq_di, Wk_di, bk_di, W1, b1, W2, b2, W3, b3, W4, b4):
    n1, n1p, w1 = N_DR_, 25088, 32
    n2, n2p, w2 = N_DR_ + N_DI_, 50176, 16

    hw_dr = emb_dr @ W_gt_dr + b_gt_dr
    hw_di = emb_di @ W_gt_di + b_gt_di

    e_drdr_s = _pad_edges(drdr_similarity_graph, n1)
    e_drdr_d = _pad_edges(drdr_dissimilarity_graph, n1)
    e_didi_s = _pad_edges(didi_similarity_graph, n1)
    e_didi_d = _pad_edges(didi_dissimilarity_graph, n1)
    e_pos = _pad_edges(positive_heterograph, n2)
    e_neg = _pad_edges(negative_heterograph, n2)

    dr_sim_p, _ = _graph_pass(e_drdr_s, hw_dr, n1, n1p, w1)
    dr_sim_n, _ = _graph_pass(e_drdr_d, hw_dr, n1, n1p, w1)
    di_sim_p, _ = _graph_pass(e_didi_s, hw_di, n1, n1p, w1)
    di_sim_n, _ = _graph_pass(e_didi_d, hw_di, n1, n1p, w1)

    drug_h = drug_feature @ W_drug_lin + b_drug_lin
    dis_h = disease_feature @ W_dis_lin + b_dis_lin
    feat0 = jnp.concatenate([drug_h, dis_h], axis=0)

    fw0 = feat0 @ W_hgt + b_hgt
    f1p, deg_p = _graph_pass(e_pos, fw0, n2, n2p, w2)
    f1n, deg_n = _graph_pass(e_neg, fw0, n2, n2p, w2)
    fw1p = f1p[:n2] @ W_hgt + b_hgt
    fw1n = f1n[:n2] @ W_hgt + b_hgt
    f2p, _ = _graph_pass(e_pos, fw1p, n2, n2p, w2, deg=deg_p)
    f2n, _ = _graph_pass(e_neg, fw1n, n2, n2p, w2, deg=deg_n)

    dr = jnp.stack([dr_sim_p[:n1], dr_sim_n[:n1],
                    f2p[:N_DR_], f2n[:N_DR_]], axis=1)
    di = jnp.stack([di_sim_p[:n1], di_sim_n[:n1],
                    f2p[N_DR_:n2], f2n[N_DR_:n2]], axis=1)
    dr_final = _self_att(dr, Wq_dr, bq_dr, Wk_dr, bk_dr)
    di_final = _self_att(di, Wq_di, bq_di, Wk_di, bk_di)

    dr_s = jnp.take(dr_final, sample[:, 0], axis=0)
    di_s = jnp.take(di_final, sample[:, 1], axis=0)
    m_result = dr_s * di_s
    r_result = _rotate(dr_s, di_s)
    drdi = jnp.concatenate([dr_s, di_s, m_result, r_result], axis=1)
    h = jax.nn.relu(drdi @ W1 + b1)
    h = jax.nn.relu(h @ W2 + b2)
    h = jax.nn.relu(h @ W3 + b3)
    return h @ W4 + b4


# speedup vs baseline: 2.7946x; 1.0005x over previous
import functools

import jax
import jax.numpy as jnp
from jax import lax
from jax.experimental import pallas as pl
from jax.experimental.pallas import tpu as pltpu
from jax.experimental.pallas import tpu_sc as plsc

N_DR_ = 25000
N_DI_ = 25000
E_ = 400000
D_ = 128
H_ = 8
B_ = 16384

_EB = 128
_NBLK = 3200
_BPW = _NBLK // 32
_ZCH = 112
_NBUF = 10
_DIAG_NO_SCATTER = True


def _segsum_call(n_pad, w, n_chunks, with_deg, src3, dst3, tables):
    C = n_chunks + (1 if with_deg else 0)
    rows_per = n_pad // 16
    assert rows_per % _ZCH == 0
    mesh = plsc.VectorSubcoreMesh(core_axis_name="c", subcore_axis_name="s")

    @functools.partial(
        pl.kernel,
        mesh=mesh,
        compiler_params=pltpu.CompilerParams(use_tc_tiling_on_sc=False),
        out_type=jax.ShapeDtypeStruct((2, C, n_pad, w), jnp.float32),
        scratch_types=[
            pltpu.VMEM((_BPW, _EB), jnp.int32),
            pltpu.VMEM((_BPW, _EB), jnp.int32),
            pltpu.VMEM((_NBUF, _EB, w), jnp.float32),
            pltpu.VMEM((_ZCH, w), jnp.float32),
            pltpu.VMEM((_EB, w), jnp.float32),
            pltpu.VMEM_SHARED((n_pad, w), jnp.float32),
            pltpu.SemaphoreType.DMA((_NBUF,)),
            pltpu.SemaphoreType.DMA((_NBUF,)),
        ],
    )
    def k(src_h, dst_h, *rest):
        tabs = rest[:n_chunks]
        zrow_h = rest[n_chunks]
        ones_h = rest[n_chunks + 1]
        out_h = rest[n_chunks + 2]
        src_v, dst_v, rows_v, zbuf, obuf, acc, gsem, ssem = rest[n_chunks + 3:]
        cid = lax.axis_index("c")
        sid = lax.axis_index("s")
        wid = cid * 16 + sid
        pltpu.sync_copy(src_h.at[wid], src_v)
        pltpu.sync_copy(dst_h.at[wid], dst_v)
        pltpu.sync_copy(zrow_h, zbuf)
        pltpu.sync_copy(ones_h, obuf)
        r0 = sid * rows_per
        for c in range(C):
            @pl.loop(0, rows_per, step=_ZCH)
            def _(rz):
                pltpu.sync_copy(zbuf, acc.at[pl.ds(r0 + rz, _ZCH)])
            plsc.subcore_barrier()
            if c < n_chunks:
                def _g_start(b, i):
                    pltpu.async_copy(tabs[c].at[src_v.at[b]],
                                     rows_v.at[i], gsem.at[i])

                def _g_wait(b, i):
                    pltpu.make_async_copy(tabs[c].at[src_v.at[b]],
                                          rows_v.at[i], gsem.at[i]).wait()

                def _s_start(b, i):
                    pltpu.async_copy(rows_v.at[i], acc.at[dst_v.at[b]],
                                     ssem.at[i], add=True)

                def _s_wait(b, i):
                    pltpu.make_async_copy(rows_v.at[i], acc.at[dst_v.at[b]],
                                          ssem.at[i]).wait()

                for i in range(_NBUF):
                    _g_start(i, i)

                @pl.loop(0, _BPW, step=_NBUF)
                def _(g):
                    for i in range(_NBUF):
                        _g_wait(g + i, i)
                        if not _DIAG_NO_SCATTER:
                            _s_start(g + i, i)
                    for i in range(_NBUF):
                        if not _DIAG_NO_SCATTER:
                            _s_wait(g + i, i)
                        nb = jnp.minimum(g + _NBUF + i, _BPW - 1)
                        _g_start(nb, i)
                for i in range(_NBUF):
                    _g_wait(_BPW - 1, i)
            else:
                def _d_start(b, i):
                    pltpu.async_copy(obuf, acc.at[dst_v.at[b]],
                                     ssem.at[i], add=True)

                def _d_wait(b, i):
                    pltpu.make_async_copy(obuf, acc.at[dst_v.at[b]],
                                          ssem.at[i]).wait()

                for i in range(_NBUF):
                    _d_start(i, i)

                @pl.loop(_NBUF, _BPW, step=_NBUF)
                def _(g):
                    for i in range(_NBUF):
                        _d_wait(g - _NBUF + i, i)
                        _d_start(g + i, i)
                for i in range(_NBUF):
                    _d_wait(_BPW - _NBUF + i, i)
            plsc.subcore_barrier()
            pltpu.sync_copy(acc.at[pl.ds(r0, rows_per)],
                            out_h.at[cid, c, pl.ds(r0, rows_per)])
            plsc.subcore_barrier()

    zrow = jnp.zeros((_ZCH, w), jnp.float32)
    ones = jnp.ones((_EB, w), jnp.float32)
    return k(src3, dst3, *tables, zrow, ones)


def _pad_edges(e, n):
    pad = jnp.full((2, _NBLK * _EB - E_), n, jnp.int32)
    return jnp.concatenate([e, pad], axis=1).reshape(2, 32, _BPW, _EB)


def _chunk_table(hw, n_pad, w):
    n = hw.shape[0]
    hwp = jnp.pad(hw, ((0, n_pad - n), (0, 0)))
    return [hwp[:, i * w:(i + 1) * w] for i in range(D_ // w)]


def _graph_pass(edges3, hw, n, n_pad, w, deg=None):
    tables = _chunk_table(hw, n_pad, w)
    with_deg = deg is None
    parts = _segsum_call(n_pad, w, len(tables), with_deg,
                         edges3[0], edges3[1], tables)
    sums = parts[0] + parts[1]
    agg = jnp.concatenate([sums[c] for c in range(len(tables))], axis=1)
    if with_deg:
        deg = sums[len(tables), :, 0]
    res = jax.nn.relu(agg / jnp.maximum(deg, 1.0)[:, None])
    return res, deg


def _self_att(x, Wq, bq, Wk, bk):
    Bn, M, Cc = x.shape
    Dh = Cc // H_
    q = (jnp.mean(x, axis=1) @ Wq + bq).reshape(Bn, 1, H_, Dh).transpose(0, 2, 1, 3)
    k = (x @ Wk + bk).reshape(Bn, M, H_, Dh).transpose(0, 2, 3, 1)
    v = x.reshape(Bn, M, H_, Dh).transpose(0, 2, 1, 3)
    alpha = jax.nn.softmax((q @ k) / (float(Dh) ** 0.5), axis=-1)
    o = alpha @ v
    return o.transpose(0, 2, 1, 3).reshape(Bn, H_ * Dh)


def _rotate(a, b):
    a_re, a_im = jnp.split(a, 2, axis=-1)
    b_re, b_im = jnp.split(b, 2, axis=-1)
    return jnp.concatenate([a_re * b_re - a_im * b_im,
                            a_re * b_im + a_im * b_re], axis=-1)


def kernel(drdr_similarity_graph, didi_similarity_graph, drdr_dissimilarity_graph, didi_dissimilarity_graph, positive_heterograph, negative_heterograph, drug_feature, disease_feature, sample, emb_dr, emb_di, W_gt_dr, b_gt_dr, W_gt_di, b_gt_di, W_drug_lin, b_drug_lin, W_dis_lin, b_dis_lin, W_hgt, b_hgt, Wq_dr, bq_dr, Wk_dr, bk_dr, Wq_di, bq_di, Wk_di, bk_di, W1, b1, W2, b2, W3, b3, W4, b4):
    n1, n1p, w1 = N_DR_, 25088, 32
    n2, n2p, w2 = N_DR_ + N_DI_, 50176, 16

    hw_dr = emb_dr @ W_gt_dr + b_gt_dr
    hw_di = emb_di @ W_gt_di + b_gt_di

    e_drdr_s = _pad_edges(drdr_similarity_graph, n1)
    e_drdr_d = _pad_edges(drdr_dissimilarity_graph, n1)
    e_didi_s = _pad_edges(didi_similarity_graph, n1)
    e_didi_d = _pad_edges(didi_dissimilarity_graph, n1)
    e_pos = _pad_edges(positive_heterograph, n2)
    e_neg = _pad_edges(negative_heterograph, n2)

    dr_sim_p, _ = _graph_pass(e_drdr_s, hw_dr, n1, n1p, w1)
    dr_sim_n, _ = _graph_pass(e_drdr_d, hw_dr, n1, n1p, w1)
    di_sim_p, _ = _graph_pass(e_didi_s, hw_di, n1, n1p, w1)
    di_sim_n, _ = _graph_pass(e_didi_d, hw_di, n1, n1p, w1)

    drug_h = drug_feature @ W_drug_lin + b_drug_lin
    dis_h = disease_feature @ W_dis_lin + b_dis_lin
    feat0 = jnp.concatenate([drug_h, dis_h], axis=0)

    fw0 = feat0 @ W_hgt + b_hgt
    f1p, deg_p = _graph_pass(e_pos, fw0, n2, n2p, w2)
    f1n, deg_n = _graph_pass(e_neg, fw0, n2, n2p, w2)
    fw1p = f1p[:n2] @ W_hgt + b_hgt
    fw1n = f1n[:n2] @ W_hgt + b_hgt
    f2p, _ = _graph_pass(e_pos, fw1p, n2, n2p, w2, deg=deg_p)
    f2n, _ = _graph_pass(e_neg, fw1n, n2, n2p, w2, deg=deg_n)

    dr = jnp.stack([dr_sim_p[:n1], dr_sim_n[:n1],
                    f2p[:N_DR_], f2n[:N_DR_]], axis=1)
    di = jnp.stack([di_sim_p[:n1], di_sim_n[:n1],
                    f2p[N_DR_:n2], f2n[N_DR_:n2]], axis=1)
    dr_final = _self_att(dr, Wq_dr, bq_dr, Wk_dr, bk_dr)
    di_final = _self_att(di, Wq_di, bq_di, Wk_di, bk_di)

    dr_s = jnp.take(dr_final, sample[:, 0], axis=0)
    di_s = jnp.take(di_final, sample[:, 1], axis=0)
    m_result = dr_s * di_s
    r_result = _rotate(dr_s, di_s)
    drdi = jnp.concatenate([dr_s, di_s, m_result, r_result], axis=1)
    h = jax.nn.relu(drdi @ W1 + b1)
    h = jax.nn.relu(h @ W2 + b2)
    h = jax.nn.relu(h @ W3 + b3)
    return h @ W4 + b4


# speedup vs baseline: 3.4350x; 1.2291x over previous
import functools

import jax
import jax.numpy as jnp
from jax import lax
from jax.experimental import pallas as pl
from jax.experimental.pallas import tpu as pltpu
from jax.experimental.pallas import tpu_sc as plsc

N_DR_ = 25000
N_DI_ = 25000
E_ = 400000
D_ = 128
H_ = 8
B_ = 16384

_EB = 128
_NBLK = 3200
_BPW = _NBLK // 32
_ZCH = 112
_NBUF = 10
_DIAG_NO_SCATTER = False


def _segsum_call(n_pad, w, n_chunks, with_deg, src3, dst3, tables):
    C = n_chunks + (1 if with_deg else 0)
    rows_per = n_pad // 16
    assert rows_per % _ZCH == 0
    mesh = plsc.VectorSubcoreMesh(core_axis_name="c", subcore_axis_name="s")

    @functools.partial(
        pl.kernel,
        mesh=mesh,
        compiler_params=pltpu.CompilerParams(use_tc_tiling_on_sc=False),
        out_type=jax.ShapeDtypeStruct((2, C, n_pad, w), jnp.float32),
        scratch_types=[
            pltpu.VMEM((_BPW, _EB), jnp.int32),
            pltpu.VMEM((_BPW, _EB), jnp.int32),
            pltpu.VMEM((_NBUF, _EB, w), jnp.float32),
            pltpu.VMEM((_ZCH, w), jnp.float32),
            pltpu.VMEM((_EB, w), jnp.float32),
            pltpu.VMEM_SHARED((n_pad, w), jnp.float32),
            pltpu.SemaphoreType.DMA((_NBUF,)),
            pltpu.SemaphoreType.DMA((_NBUF,)),
        ],
    )
    def k(src_h, dst_h, *rest):
        tabs = rest[:n_chunks]
        zrow_h = rest[n_chunks]
        ones_h = rest[n_chunks + 1]
        out_h = rest[n_chunks + 2]
        src_v, dst_v, rows_v, zbuf, obuf, acc, gsem, ssem = rest[n_chunks + 3:]
        cid = lax.axis_index("c")
        sid = lax.axis_index("s")
        wid = cid * 16 + sid
        pltpu.sync_copy(src_h.at[wid], src_v)
        pltpu.sync_copy(dst_h.at[wid], dst_v)
        pltpu.sync_copy(zrow_h, zbuf)
        pltpu.sync_copy(ones_h, obuf)
        r0 = sid * rows_per
        for c in range(C):
            @pl.loop(0, rows_per, step=_ZCH)
            def _(rz):
                pltpu.sync_copy(zbuf, acc.at[pl.ds(r0 + rz, _ZCH)])
            plsc.subcore_barrier()
            if c < n_chunks:
                def _g_start(b, i):
                    pltpu.async_copy(tabs[c].at[src_v.at[b]],
                                     rows_v.at[i], gsem.at[i])

                def _g_wait(b, i):
                    pltpu.make_async_copy(tabs[c].at[src_v.at[b]],
                                          rows_v.at[i], gsem.at[i]).wait()

                def _s_start(b, i):
                    pltpu.async_copy(rows_v.at[i], acc.at[dst_v.at[b]],
                                     ssem.at[i], add=True)

                def _s_wait(b, i):
                    pltpu.make_async_copy(rows_v.at[i], acc.at[dst_v.at[b]],
                                          ssem.at[i]).wait()

                for i in range(_NBUF):
                    _g_start(i, i)

                @pl.loop(0, _BPW, step=_NBUF)
                def _(g):
                    for i in range(_NBUF):
                        _g_wait(g + i, i)
                        if not _DIAG_NO_SCATTER:
                            _s_start(g + i, i)
                    for i in range(_NBUF):
                        if not _DIAG_NO_SCATTER:
                            _s_wait(g + i, i)
                        nb = jnp.minimum(g + _NBUF + i, _BPW - 1)
                        _g_start(nb, i)
                for i in range(_NBUF):
                    _g_wait(_BPW - 1, i)
            else:
                def _d_start(b, i):
                    pltpu.async_copy(obuf, acc.at[dst_v.at[b]],
                                     ssem.at[i], add=True)

                def _d_wait(b, i):
                    pltpu.make_async_copy(obuf, acc.at[dst_v.at[b]],
                                          ssem.at[i]).wait()

                for i in range(_NBUF):
                    _d_start(i, i)

                @pl.loop(_NBUF, _BPW, step=_NBUF)
                def _(g):
                    for i in range(_NBUF):
                        _d_wait(g - _NBUF + i, i)
                        _d_start(g + i, i)
                for i in range(_NBUF):
                    _d_wait(_BPW - _NBUF + i, i)
            plsc.subcore_barrier()
            pltpu.sync_copy(acc.at[pl.ds(r0, rows_per)],
                            out_h.at[cid, c, pl.ds(r0, rows_per)])
            plsc.subcore_barrier()

    zrow = jnp.zeros((_ZCH, w), jnp.float32)
    ones = jnp.ones((_EB, w), jnp.float32)
    return k(src3, dst3, *tables, zrow, ones)


_RT = 784


def _mm(x, W, b):
    N, K = x.shape

    def body(x_ref, w_ref, b_ref, o_ref):
        o_ref[...] = jnp.dot(x_ref[...], w_ref[...],
                             preferred_element_type=jnp.float32) + b_ref[...]

    return pl.pallas_call(
        body,
        grid=(N // _RT,),
        in_specs=[pl.BlockSpec((_RT, K), lambda i: (i, 0)),
                  pl.BlockSpec((K, 128), lambda i: (0, 0)),
                  pl.BlockSpec((1, 128), lambda i: (0, 0))],
        out_specs=pl.BlockSpec((_RT, 128), lambda i: (i, 0)),
        out_shape=jax.ShapeDtypeStruct((N, 128), jnp.float32),
    )(x, W, b.reshape(1, 128))


def _combine(parts, deg, W=None, b=None):
    _, C, n_pad, w = parts.shape
    nc = D_ // w
    has_w = W is not None

    def body(p_ref, d_ref, *rest):
        o_ref = rest[-1]
        s = p_ref[0] + p_ref[1]
        agg = jnp.concatenate([s[c] for c in range(nc)], axis=-1)
        dg = jnp.maximum(d_ref[...], 1.0)
        r = jax.nn.relu(agg / dg)
        if has_w:
            r = jnp.dot(r, rest[0][...],
                        preferred_element_type=jnp.float32) + rest[1][...]
        o_ref[...] = r

    in_specs = [pl.BlockSpec((2, C, _RT, w), lambda i: (0, 0, i, 0)),
                pl.BlockSpec((_RT, 1), lambda i: (i, 0))]
    args = [parts, deg.reshape(n_pad, 1)]
    if has_w:
        in_specs += [pl.BlockSpec((128, 128), lambda i: (0, 0)),
                     pl.BlockSpec((1, 128), lambda i: (0, 0))]
        args += [W, b.reshape(1, 128)]
    return pl.pallas_call(
        body,
        grid=(n_pad // _RT,),
        in_specs=in_specs,
        out_specs=pl.BlockSpec((_RT, 128), lambda i: (i, 0)),
        out_shape=jax.ShapeDtypeStruct((n_pad, 128), jnp.float32),
    )(*args)


def _attn(xs, Wq, bq, Wk, bk):
    n_pad = xs[0].shape[0]
    Dh = D_ // H_
    S = jnp.repeat(jnp.eye(H_, dtype=jnp.float32), Dh, axis=0)
    ST = jnp.repeat(jnp.eye(H_, dtype=jnp.float32), Dh, axis=1)

    def body(x0, x1, x2, x3, wq, bqr, wk, bkr, s_ref, st_ref, o_ref):
        xm = (x0[...] + x1[...] + x2[...] + x3[...]) * 0.25
        q = jnp.dot(xm, wq[...], preferred_element_type=jnp.float32) + bqr[...]
        s_mat = s_ref[...]
        scores = []
        for xr in (x0, x1, x2, x3):
            km = jnp.dot(xr[...], wk[...],
                         preferred_element_type=jnp.float32) + bkr[...]
            scores.append(jnp.dot(q * km, s_mat,
                                  preferred_element_type=jnp.float32) * 0.25)
        mx = jnp.maximum(jnp.maximum(scores[0], scores[1]),
                         jnp.maximum(scores[2], scores[3]))
        es = [jnp.exp(sc - mx) for sc in scores]
        den = es[0] + es[1] + es[2] + es[3]
        out = jnp.zeros_like(x0[...])
        for e, xr in zip(es, (x0, x1, x2, x3)):
            a = e / den
            ab = jnp.dot(a, st_ref[...], preferred_element_type=jnp.float32)
            out = out + ab * xr[...]
        o_ref[...] = out

    rs = pl.BlockSpec((_RT, 128), lambda i: (i, 0))
    ws = pl.BlockSpec((128, 128), lambda i: (0, 0))
    bs = pl.BlockSpec((1, 128), lambda i: (0, 0))
    return pl.pallas_call(
        body,
        grid=(n_pad // _RT,),
        in_specs=[rs, rs, rs, rs, ws, bs, ws, bs,
                  pl.BlockSpec((128, H_), lambda i: (0, 0)),
                  pl.BlockSpec((H_, 128), lambda i: (0, 0))],
        out_specs=rs,
        out_shape=jax.ShapeDtypeStruct((n_pad, 128), jnp.float32),
    )(*xs, Wq, bq.reshape(1, 128), Wk, bk.reshape(1, 128), S, ST)


def _gather_pairs(dr_tab, di_tab, idx0, idx1):
    mesh = plsc.VectorSubcoreMesh(core_axis_name="c", subcore_axis_name="s")
    blocks = B_ // 32 // _EB

    @functools.partial(
        pl.kernel,
        mesh=mesh,
        compiler_params=pltpu.CompilerParams(use_tc_tiling_on_sc=False),
        out_type=(jax.ShapeDtypeStruct((B_, 128), jnp.float32),
                  jax.ShapeDtypeStruct((B_, 128), jnp.float32)),
        scratch_types=[
            pltpu.VMEM((blocks, _EB), jnp.int32),
            pltpu.VMEM((blocks, _EB), jnp.int32),
            pltpu.VMEM((2, _EB, 128), jnp.float32),
            pltpu.SemaphoreType.DMA((2,)),
        ],
    )
    def k(drt, dit, i0_h, i1_h, o0_h, o1_h, i0_v, i1_v, rows, sem):
        cid = lax.axis_index("c")
        sid = lax.axis_index("s")
        wid = cid * 16 + sid
        pltpu.sync_copy(i0_h.at[wid], i0_v)
        pltpu.sync_copy(i1_h.at[wid], i1_v)
        base = wid * blocks

        @pl.loop(0, blocks)
        def _(bl):
            g0 = pltpu.async_copy(drt.at[i0_v.at[bl]], rows.at[0], sem.at[0])
            g1 = pltpu.async_copy(dit.at[i1_v.at[bl]], rows.at[1], sem.at[1])
            g0.wait()
            pltpu.sync_copy(rows.at[0], o0_h.at[pl.ds((base + bl) * _EB, _EB)])
            g1.wait()
            pltpu.sync_copy(rows.at[1], o1_h.at[pl.ds((base + bl) * _EB, _EB)])

    i0 = idx0.reshape(32, blocks, _EB)
    i1 = idx1.reshape(32, blocks, _EB)
    return k(dr_tab, di_tab, i0, i1)


def _mlp(dr_s, di_s, W1, b1, W2, b2, W3, b3, W4, b4):
    W4p = jnp.pad(W4, ((0, 0), (0, 126)))
    b4p = jnp.pad(b4, (0, 126))
    bt = 512

    def body(dr_ref, di_ref, w1, b1r, w2, b2r, w3, b3r, w4, b4r, o_ref):
        dr = dr_ref[...]
        di = di_ref[...]
        dre, dri = dr[:, :64], dr[:, 64:]
        die, dii = di[:, :64], di[:, 64:]
        rot = jnp.concatenate([dre * die - dri * dii,
                               dre * dii + dri * die], axis=-1)
        x = jnp.concatenate([dr, di, dr * di, rot], axis=-1)
        h = jax.nn.relu(jnp.dot(x, w1[...],
                                preferred_element_type=jnp.float32) + b1r[...])
        h = jax.nn.relu(jnp.dot(h, w2[...],
                                preferred_element_type=jnp.float32) + b2r[...])
        h = jax.nn.relu(jnp.dot(h, w3[...],
                                preferred_element_type=jnp.float32) + b3r[...])
        o_ref[...] = jnp.dot(h, w4[...],
                             preferred_element_type=jnp.float32) + b4r[...]

    rs = pl.BlockSpec((bt, 128), lambda i: (i, 0))

    def fullspec(shape):
        return pl.BlockSpec(shape, lambda i: tuple(0 for _ in shape))

    return pl.pallas_call(
        body,
        grid=(B_ // bt,),
        in_specs=[rs, rs,
                  fullspec((512, 1024)), fullspec((1, 1024)),
                  fullspec((1024, 1024)), fullspec((1, 1024)),
                  fullspec((1024, 256)), fullspec((1, 256)),
                  fullspec((256, 128)), fullspec((1, 128))],
        out_specs=rs,
        out_shape=jax.ShapeDtypeStruct((B_, 128), jnp.float32),
    )(dr_s, di_s, W1, b1.reshape(1, -1), W2, b2.reshape(1, -1),
      W3, b3.reshape(1, -1), W4p, b4p.reshape(1, -1))


def _pad_edges(e, n):
    pad = jnp.full((2, _NBLK * _EB - E_), n, jnp.int32)
    return jnp.concatenate([e, pad], axis=1).reshape(2, 32, _BPW, _EB)


def _graph_pass(edges3, hw, n_pad, w, deg=None):
    tables = [hw[:, i * w:(i + 1) * w] for i in range(D_ // w)]
    with_deg = deg is None
    parts = _segsum_call(n_pad, w, len(tables), with_deg,
                         edges3[0], edges3[1], tables)
    if with_deg:
        C = len(tables)
        deg = parts[0, C, :, 0] + parts[1, C, :, 0]
    return parts, deg


def _self_att(x, Wq, bq, Wk, bk):
    Bn, M, Cc = x.shape
    Dh = Cc // H_
    q = (jnp.mean(x, axis=1) @ Wq + bq).reshape(Bn, 1, H_, Dh).transpose(0, 2, 1, 3)
    k = (x @ Wk + bk).reshape(Bn, M, H_, Dh).transpose(0, 2, 3, 1)
    v = x.reshape(Bn, M, H_, Dh).transpose(0, 2, 1, 3)
    alpha = jax.nn.softmax((q @ k) / (float(Dh) ** 0.5), axis=-1)
    o = alpha @ v
    return o.transpose(0, 2, 1, 3).reshape(Bn, H_ * Dh)


def _rotate(a, b):
    a_re, a_im = jnp.split(a, 2, axis=-1)
    b_re, b_im = jnp.split(b, 2, axis=-1)
    return jnp.concatenate([a_re * b_re - a_im * b_im,
                            a_re * b_im + a_im * b_re], axis=-1)


def kernel(drdr_similarity_graph, didi_similarity_graph, drdr_dissimilarity_graph, didi_dissimilarity_graph, positive_heterograph, negative_heterograph, drug_feature, disease_feature, sample, emb_dr, emb_di, W_gt_dr, b_gt_dr, W_gt_di, b_gt_di, W_drug_lin, b_drug_lin, W_dis_lin, b_dis_lin, W_hgt, b_hgt, Wq_dr, bq_dr, Wk_dr, bk_dr, Wq_di, bq_di, Wk_di, bk_di, W1, b1, W2, b2, W3, b3, W4, b4):
    n1, n1p, w1 = N_DR_, 25088, 32
    n2, n2p, w2 = N_DR_ + N_DI_, 50176, 16

    emb_dr_p = jnp.pad(emb_dr, ((0, n1p - n1), (0, 0)))
    emb_di_p = jnp.pad(emb_di, ((0, n1p - n1), (0, 0)))
    hw_dr = _mm(emb_dr_p, W_gt_dr, b_gt_dr)
    hw_di = _mm(emb_di_p, W_gt_di, b_gt_di)

    e_drdr_s = _pad_edges(drdr_similarity_graph, n1)
    e_drdr_d = _pad_edges(drdr_dissimilarity_graph, n1)
    e_didi_s = _pad_edges(didi_similarity_graph, n1)
    e_didi_d = _pad_edges(didi_dissimilarity_graph, n1)
    e_pos = _pad_edges(positive_heterograph, n2)
    e_neg = _pad_edges(negative_heterograph, n2)

    p_drs, d_drs = _graph_pass(e_drdr_s, hw_dr, n1p, w1)
    p_drd, d_drd = _graph_pass(e_drdr_d, hw_dr, n1p, w1)
    p_dis, d_dis = _graph_pass(e_didi_s, hw_di, n1p, w1)
    p_did, d_did = _graph_pass(e_didi_d, hw_di, n1p, w1)
    dr_sim_p = _combine(p_drs, d_drs)
    dr_sim_n = _combine(p_drd, d_drd)
    di_sim_p = _combine(p_dis, d_dis)
    di_sim_n = _combine(p_did, d_did)

    drug_p = jnp.pad(drug_feature, ((0, n1p - n1), (0, 4)))
    W_drug_p = jnp.pad(W_drug_lin, ((0, 4), (0, 0)))
    drug_h = _mm(drug_p, W_drug_p, b_drug_lin)
    dis_p = jnp.pad(disease_feature, ((0, n1p - n1), (0, 0)))
    dis_h = _mm(dis_p, W_dis_lin, b_dis_lin)
    feat0p = jnp.pad(jnp.concatenate([drug_h[:n1], dis_h[:N_DI_]], axis=0),
                     ((0, n2p - n2), (0, 0)))
    fw0 = _mm(feat0p, W_hgt, b_hgt)

    pp1, deg_p = _graph_pass(e_pos, fw0, n2p, w2)
    pn1, deg_n = _graph_pass(e_neg, fw0, n2p, w2)
    fw1p = _combine(pp1, deg_p, W_hgt, b_hgt)
    fw1n = _combine(pn1, deg_n, W_hgt, b_hgt)
    pp2, _ = _graph_pass(e_pos, fw1p, n2p, w2, deg=deg_p)
    pn2, _ = _graph_pass(e_neg, fw1n, n2p, w2, deg=deg_n)
    f2p = _combine(pp2, deg_p)
    f2n = _combine(pn2, deg_n)

    dr_final = _attn([dr_sim_p, dr_sim_n, f2p[:n1p], f2n[:n1p]],
                     Wq_dr, bq_dr, Wk_dr, bk_dr)
    di_final = _attn([di_sim_p, di_sim_n,
                      f2p[N_DR_:N_DR_ + n1p], f2n[N_DR_:N_DR_ + n1p]],
                     Wq_di, bq_di, Wk_di, bk_di)

    dr_s, di_s = _gather_pairs(dr_final, di_final, sample[:, 0], sample[:, 1])
    out = _mlp(dr_s, di_s, W1, b1, W2, b2, W3, b3, W4, b4)
    return out[:, :2]


# speedup vs baseline: 3.4359x; 1.0003x over previous
import functools

import jax
import jax.numpy as jnp
from jax import lax
from jax.experimental import pallas as pl
from jax.experimental.pallas import tpu as pltpu
from jax.experimental.pallas import tpu_sc as plsc

N_DR_ = 25000
N_DI_ = 25000
E_ = 400000
D_ = 128
H_ = 8
B_ = 16384

_EB = 128
_NBLK = 3200
_BPW = _NBLK // 32
_ZCH = 112
_NBUF = 10


def _segsum_call(n_pad, w, n_chunks, with_deg, src3, dst3, tables):
    C = n_chunks + (1 if with_deg else 0)
    rows_per = n_pad // 16
    assert rows_per % _ZCH == 0
    mesh = plsc.VectorSubcoreMesh(core_axis_name="c", subcore_axis_name="s")

    @functools.partial(
        pl.kernel,
        mesh=mesh,
        compiler_params=pltpu.CompilerParams(use_tc_tiling_on_sc=False),
        out_type=jax.ShapeDtypeStruct((2, C, n_pad, w), jnp.float32),
        scratch_types=[
            pltpu.VMEM((_BPW, _EB), jnp.int32),
            pltpu.VMEM((_BPW, _EB), jnp.int32),
            pltpu.VMEM((_NBUF, _EB, w), jnp.float32),
            pltpu.VMEM((_ZCH, w), jnp.float32),
            pltpu.VMEM((_EB, w), jnp.float32),
            pltpu.VMEM_SHARED((n_pad, w), jnp.float32),
            pltpu.SemaphoreType.DMA((_NBUF,)),
            pltpu.SemaphoreType.DMA((_NBUF,)),
        ],
    )
    def k(src_h, dst_h, *rest):
        tabs = rest[:n_chunks]
        zrow_h = rest[n_chunks]
        ones_h = rest[n_chunks + 1]
        out_h = rest[n_chunks + 2]
        src_v, dst_v, rows_v, zbuf, obuf, acc, gsem, ssem = rest[n_chunks + 3:]
        cid = lax.axis_index("c")
        sid = lax.axis_index("s")
        wid = cid * 16 + sid
        pltpu.sync_copy(src_h.at[wid], src_v)
        pltpu.sync_copy(dst_h.at[wid], dst_v)
        pltpu.sync_copy(zrow_h, zbuf)
        pltpu.sync_copy(ones_h, obuf)
        r0 = sid * rows_per
        for c in range(C):
            @pl.loop(0, rows_per, step=_ZCH)
            def _(rz):
                pltpu.sync_copy(zbuf, acc.at[pl.ds(r0 + rz, _ZCH)])
            plsc.subcore_barrier()
            if c < n_chunks:
                def _g_start(b, i):
                    pltpu.async_copy(tabs[c].at[src_v.at[b]],
                                     rows_v.at[i], gsem.at[i])

                def _g_wait(b, i):
                    pltpu.make_async_copy(tabs[c].at[src_v.at[b]],
                                          rows_v.at[i], gsem.at[i]).wait()

                def _s_start(b, i):
                    pltpu.async_copy(rows_v.at[i], acc.at[dst_v.at[b]],
                                     ssem.at[i], add=True)

                def _s_wait(b, i):
                    pltpu.make_async_copy(rows_v.at[i], acc.at[dst_v.at[b]],
                                          ssem.at[i]).wait()

                for i in range(_NBUF):
                    _g_start(i, i)

                @pl.loop(0, _BPW, step=_NBUF)
                def _(g):
                    for i in range(_NBUF):
                        _g_wait(g + i, i)
                        _s_start(g + i, i)
                    for i in range(_NBUF):
                        _s_wait(g + i, i)
                        nb = jnp.minimum(g + _NBUF + i, _BPW - 1)
                        _g_start(nb, i)
                for i in range(_NBUF):
                    _g_wait(_BPW - 1, i)
            else:
                def _d_start(b, i):
                    pltpu.async_copy(obuf, acc.at[dst_v.at[b]],
                                     ssem.at[i], add=True)

                def _d_wait(b, i):
                    pltpu.make_async_copy(obuf, acc.at[dst_v.at[b]],
                                          ssem.at[i]).wait()

                for i in range(_NBUF):
                    _d_start(i, i)

                @pl.loop(_NBUF, _BPW, step=_NBUF)
                def _(g):
                    for i in range(_NBUF):
                        _d_wait(g - _NBUF + i, i)
                        _d_start(g + i, i)
                for i in range(_NBUF):
                    _d_wait(_BPW - _NBUF + i, i)
            plsc.subcore_barrier()
            pltpu.sync_copy(acc.at[pl.ds(r0, rows_per)],
                            out_h.at[cid, c, pl.ds(r0, rows_per)])
            plsc.subcore_barrier()

    zrow = jnp.zeros((_ZCH, w), jnp.float32)
    ones = jnp.ones((_EB, w), jnp.float32)
    return k(src3, dst3, *tables, zrow, ones)


_RT = 784


def _mm(x, W, b):
    N, K = x.shape

    def body(x_ref, w_ref, b_ref, o_ref):
        o_ref[...] = jnp.dot(x_ref[...], w_ref[...],
                             preferred_element_type=jnp.float32) + b_ref[...]

    return pl.pallas_call(
        body,
        grid=(N // _RT,),
        in_specs=[pl.BlockSpec((_RT, K), lambda i: (i, 0)),
                  pl.BlockSpec((K, 128), lambda i: (0, 0)),
                  pl.BlockSpec((1, 128), lambda i: (0, 0))],
        out_specs=pl.BlockSpec((_RT, 128), lambda i: (i, 0)),
        out_shape=jax.ShapeDtypeStruct((N, 128), jnp.float32),
    )(x, W, b.reshape(1, 128))


def _combine(parts, deg, W=None, b=None):
    _, C, n_pad, w = parts.shape
    nc = D_ // w
    has_w = W is not None

    def body(p_ref, d_ref, *rest):
        o_ref = rest[-1]
        s = p_ref[0] + p_ref[1]
        agg = jnp.concatenate([s[c] for c in range(nc)], axis=-1)
        dg = jnp.maximum(d_ref[...], 1.0)
        r = jax.nn.relu(agg / dg)
        if has_w:
            r = jnp.dot(r, rest[0][...],
                        preferred_element_type=jnp.float32) + rest[1][...]
        o_ref[...] = r

    in_specs = [pl.BlockSpec((2, C, _RT, w), lambda i: (0, 0, i, 0)),
                pl.BlockSpec((_RT, 1), lambda i: (i, 0))]
    args = [parts, deg.reshape(n_pad, 1)]
    if has_w:
        in_specs += [pl.BlockSpec((128, 128), lambda i: (0, 0)),
                     pl.BlockSpec((1, 128), lambda i: (0, 0))]
        args += [W, b.reshape(1, 128)]
    return pl.pallas_call(
        body,
        grid=(n_pad // _RT,),
        in_specs=in_specs,
        out_specs=pl.BlockSpec((_RT, 128), lambda i: (i, 0)),
        out_shape=jax.ShapeDtypeStruct((n_pad, 128), jnp.float32),
    )(*args)


def _attn(xs, Wq, bq, Wk, bk):
    n_pad = xs[0].shape[0]
    Dh = D_ // H_
    S = jnp.repeat(jnp.eye(H_, dtype=jnp.float32), Dh, axis=0)
    ST = jnp.repeat(jnp.eye(H_, dtype=jnp.float32), Dh, axis=1)

    def body(x0, x1, x2, x3, wq, bqr, wk, bkr, s_ref, st_ref, o_ref):
        xm = (x0[...] + x1[...] + x2[...] + x3[...]) * 0.25
        q = jnp.dot(xm, wq[...], preferred_element_type=jnp.float32) + bqr[...]
        s_mat = s_ref[...]
        scores = []
        for xr in (x0, x1, x2, x3):
            km = jnp.dot(xr[...], wk[...],
                         preferred_element_type=jnp.float32) + bkr[...]
            scores.append(jnp.dot(q * km, s_mat,
                                  preferred_element_type=jnp.float32) * 0.25)
        mx = jnp.maximum(jnp.maximum(scores[0], scores[1]),
                         jnp.maximum(scores[2], scores[3]))
        es = [jnp.exp(sc - mx) for sc in scores]
        den = es[0] + es[1] + es[2] + es[3]
        out = jnp.zeros_like(x0[...])
        for e, xr in zip(es, (x0, x1, x2, x3)):
            a = e / den
            ab = jnp.dot(a, st_ref[...], preferred_element_type=jnp.float32)
            out = out + ab * xr[...]
        o_ref[...] = out

    rs = pl.BlockSpec((_RT, 128), lambda i: (i, 0))
    ws = pl.BlockSpec((128, 128), lambda i: (0, 0))
    bs = pl.BlockSpec((1, 128), lambda i: (0, 0))
    return pl.pallas_call(
        body,
        grid=(n_pad // _RT,),
        in_specs=[rs, rs, rs, rs, ws, bs, ws, bs,
                  pl.BlockSpec((128, H_), lambda i: (0, 0)),
                  pl.BlockSpec((H_, 128), lambda i: (0, 0))],
        out_specs=rs,
        out_shape=jax.ShapeDtypeStruct((n_pad, 128), jnp.float32),
    )(*xs, Wq, bq.reshape(1, 128), Wk, bk.reshape(1, 128), S, ST)


def _gather_pairs(dr_tab, di_tab, idx0, idx1):
    mesh = plsc.VectorSubcoreMesh(core_axis_name="c", subcore_axis_name="s")
    blocks = B_ // 32 // _EB

    @functools.partial(
        pl.kernel,
        mesh=mesh,
        compiler_params=pltpu.CompilerParams(use_tc_tiling_on_sc=False),
        out_type=(jax.ShapeDtypeStruct((B_, 128), jnp.float32),
                  jax.ShapeDtypeStruct((B_, 128), jnp.float32)),
        scratch_types=[
            pltpu.VMEM((blocks, _EB), jnp.int32),
            pltpu.VMEM((blocks, _EB), jnp.int32),
            pltpu.VMEM((2, _EB, 128), jnp.float32),
            pltpu.SemaphoreType.DMA((2,)),
        ],
    )
    def k(drt, dit, i0_h, i1_h, o0_h, o1_h, i0_v, i1_v, rows, sem):
        cid = lax.axis_index("c")
        sid = lax.axis_index("s")
        wid = cid * 16 + sid
        pltpu.sync_copy(i0_h.at[wid], i0_v)
        pltpu.sync_copy(i1_h.at[wid], i1_v)
        base = wid * blocks

        @pl.loop(0, blocks)
        def _(bl):
            g0 = pltpu.async_copy(drt.at[i0_v.at[bl]], rows.at[0], sem.at[0])
            g1 = pltpu.async_copy(dit.at[i1_v.at[bl]], rows.at[1], sem.at[1])
            g0.wait()
            pltpu.sync_copy(rows.at[0], o0_h.at[pl.ds((base + bl) * _EB, _EB)])
            g1.wait()
            pltpu.sync_copy(rows.at[1], o1_h.at[pl.ds((base + bl) * _EB, _EB)])

    i0 = idx0.reshape(32, blocks, _EB)
    i1 = idx1.reshape(32, blocks, _EB)
    return k(dr_tab, di_tab, i0, i1)


def _mlp(dr_s, di_s, W1, b1, W2, b2, W3, b3, W4, b4):
    W4p = jnp.pad(W4, ((0, 0), (0, 126)))
    b4p = jnp.pad(b4, (0, 126))
    bt = 512

    def body(dr_ref, di_ref, w1, b1r, w2, b2r, w3, b3r, w4, b4r, o_ref):
        dr = dr_ref[...]
        di = di_ref[...]
        dre, dri = dr[:, :64], dr[:, 64:]
        die, dii = di[:, :64], di[:, 64:]
        rot = jnp.concatenate([dre * die - dri * dii,
                               dre * dii + dri * die], axis=-1)
        x = jnp.concatenate([dr, di, dr * di, rot], axis=-1)
        h = jax.nn.relu(jnp.dot(x, w1[...],
                                preferred_element_type=jnp.float32) + b1r[...])
        h = jax.nn.relu(jnp.dot(h, w2[...],
                                preferred_element_type=jnp.float32) + b2r[...])
        h = jax.nn.relu(jnp.dot(h, w3[...],
                                preferred_element_type=jnp.float32) + b3r[...])
        o_ref[...] = jnp.dot(h, w4[...],
                             preferred_element_type=jnp.float32) + b4r[...]

    rs = pl.BlockSpec((bt, 128), lambda i: (i, 0))

    def fullspec(shape):
        return pl.BlockSpec(shape, lambda i: tuple(0 for _ in shape))

    return pl.pallas_call(
        body,
        grid=(B_ // bt,),
        in_specs=[rs, rs,
                  fullspec((512, 1024)), fullspec((1, 1024)),
                  fullspec((1024, 1024)), fullspec((1, 1024)),
                  fullspec((1024, 256)), fullspec((1, 256)),
                  fullspec((256, 128)), fullspec((1, 128))],
        out_specs=rs,
        out_shape=jax.ShapeDtypeStruct((B_, 128), jnp.float32),
    )(dr_s, di_s, W1, b1.reshape(1, -1), W2, b2.reshape(1, -1),
      W3, b3.reshape(1, -1), W4p, b4p.reshape(1, -1))


def _pad_edges(e, n):
    pad = jnp.full((2, _NBLK * _EB - E_), n, jnp.int32)
    return jnp.concatenate([e, pad], axis=1).reshape(2, 32, _BPW, _EB)


def _graph_pass(edges3, hw, n_pad, w, deg=None):
    tables = [hw[:, i * w:(i + 1) * w] for i in range(D_ // w)]
    with_deg = deg is None
    parts = _segsum_call(n_pad, w, len(tables), with_deg,
                         edges3[0], edges3[1], tables)
    if with_deg:
        C = len(tables)
        deg = parts[0, C, :, 0] + parts[1, C, :, 0]
    return parts, deg


def kernel(drdr_similarity_graph, didi_similarity_graph, drdr_dissimilarity_graph, didi_dissimilarity_graph, positive_heterograph, negative_heterograph, drug_feature, disease_feature, sample, emb_dr, emb_di, W_gt_dr, b_gt_dr, W_gt_di, b_gt_di, W_drug_lin, b_drug_lin, W_dis_lin, b_dis_lin, W_hgt, b_hgt, Wq_dr, bq_dr, Wk_dr, bk_dr, Wq_di, bq_di, Wk_di, bk_di, W1, b1, W2, b2, W3, b3, W4, b4):
    n1, n1p, w1 = N_DR_, 25088, 32
    n2, n2p, w2 = N_DR_ + N_DI_, 50176, 16

    emb_dr_p = jnp.pad(emb_dr, ((0, n1p - n1), (0, 0)))
    emb_di_p = jnp.pad(emb_di, ((0, n1p - n1), (0, 0)))
    hw_dr = _mm(emb_dr_p, W_gt_dr, b_gt_dr)
    hw_di = _mm(emb_di_p, W_gt_di, b_gt_di)

    e_drdr_s = _pad_edges(drdr_similarity_graph, n1)
    e_drdr_d = _pad_edges(drdr_dissimilarity_graph, n1)
    e_didi_s = _pad_edges(didi_similarity_graph, n1)
    e_didi_d = _pad_edges(didi_dissimilarity_graph, n1)
    e_pos = _pad_edges(positive_heterograph, n2)
    e_neg = _pad_edges(negative_heterograph, n2)

    p_drs, d_drs = _graph_pass(e_drdr_s, hw_dr, n1p, w1)
    p_drd, d_drd = _graph_pass(e_drdr_d, hw_dr, n1p, w1)
    p_dis, d_dis = _graph_pass(e_didi_s, hw_di, n1p, w1)
    p_did, d_did = _graph_pass(e_didi_d, hw_di, n1p, w1)
    dr_sim_p = _combine(p_drs, d_drs)
    dr_sim_n = _combine(p_drd, d_drd)
    di_sim_p = _combine(p_dis, d_dis)
    di_sim_n = _combine(p_did, d_did)

    drug_p = jnp.pad(drug_feature, ((0, n1p - n1), (0, 4)))
    W_drug_p = jnp.pad(W_drug_lin, ((0, 4), (0, 0)))
    drug_h = _mm(drug_p, W_drug_p, b_drug_lin)
    dis_p = jnp.pad(disease_feature, ((0, n1p - n1), (0, 0)))
    dis_h = _mm(dis_p, W_dis_lin, b_dis_lin)
    feat0p = jnp.pad(jnp.concatenate([drug_h[:n1], dis_h[:N_DI_]], axis=0),
                     ((0, n2p - n2), (0, 0)))
    fw0 = _mm(feat0p, W_hgt, b_hgt)

    pp1, deg_p = _graph_pass(e_pos, fw0, n2p, w2)
    pn1, deg_n = _graph_pass(e_neg, fw0, n2p, w2)
    fw1p = _combine(pp1, deg_p, W_hgt, b_hgt)
    fw1n = _combine(pn1, deg_n, W_hgt, b_hgt)
    pp2, _ = _graph_pass(e_pos, fw1p, n2p, w2, deg=deg_p)
    pn2, _ = _graph_pass(e_neg, fw1n, n2p, w2, deg=deg_n)
    f2p = _combine(pp2, deg_p)
    f2n = _combine(pn2, deg_n)

    dr_final = _attn([dr_sim_p, dr_sim_n, f2p[:n1p], f2n[:n1p]],
                     Wq_dr, bq_dr, Wk_dr, bk_dr)
    di_final = _attn([di_sim_p, di_sim_n,
                      f2p[N_DR_:N_DR_ + n1p], f2n[N_DR_:N_DR_ + n1p]],
                     Wq_di, bq_di, Wk_di, bk_di)

    dr_s, di_s = _gather_pairs(dr_final, di_final, sample[:, 0], sample[:, 1])
    out = _mlp(dr_s, di_s, W1, b1, W2, b2, W3, b3, W4, b4)
    return out[:, :2]
